# baseline (speedup 1.0000x reference)
import functools

import jax
import jax.numpy as jnp
from jax import lax
from jax.experimental import pallas as pl
from jax.experimental.pallas import tpu as pltpu
from jax.experimental.pallas import tpu_sc as plsc

N = 10000
E = 320000
D = 128

NC = 2
NS = 16
NW = NC * NS
CHUNK = 128
CH = 80
E_PAD = NW * CH * CHUNK
PAIRS = CH // 2
N_PAD = 10112
ROWS_PER_TILE = N_PAD // NS
TRASH = N

_mesh = plsc.VectorSubcoreMesh(core_axis_name="c", subcore_axis_name="s")


def _sc_agg_body(h_hbm, src3, dst3, z128, agg_out,
                 sidx, didx, rows_v, agg_sh, sem):
    c = lax.axis_index("c")
    s = lax.axis_index("s")
    w = s * NC + c
    r0 = s * ROWS_PER_TILE
    pltpu.sync_copy(z128.at[pl.ds(r0, ROWS_PER_TILE)],
                    agg_sh.at[pl.ds(r0, ROWS_PER_TILE)])
    plsc.subcore_barrier()

    def chunk(j, carry):
        pltpu.sync_copy(src3.at[w, j], sidx)
        pltpu.sync_copy(dst3.at[w, j], didx)
        pltpu.async_copy(h_hbm.at[sidx], rows_v, sem).wait()
        pltpu.sync_copy(rows_v, agg_sh.at[didx], add=True)
        return carry

    lax.fori_loop(0, CH, chunk, 0)
    plsc.subcore_barrier()
    pltpu.sync_copy(agg_sh.at[pl.ds(r0, ROWS_PER_TILE)],
                    agg_out.at[c, pl.ds(r0, ROWS_PER_TILE)])


def _sc_deg_body(dst3, z128, ones_hbm, deg_out,
                 dslab, ones_v, deg_sh, semsc0, semsc1):
    c = lax.axis_index("c")
    s = lax.axis_index("s")
    w = s * NC + c
    r0 = s * ROWS_PER_TILE
    pltpu.sync_copy(z128.at[pl.ds(r0, ROWS_PER_TILE)],
                    deg_sh.at[pl.ds(r0, ROWS_PER_TILE)])
    pltpu.sync_copy(ones_hbm, ones_v)
    pltpu.sync_copy(dst3.at[w], dslab)
    plsc.subcore_barrier()

    def sc_start(j, sem):
        pltpu.async_copy(ones_v, deg_sh.at[dslab.at[j]], sem, add=True)

    def sc_wait(j, sem):
        pltpu.make_async_copy(ones_v, deg_sh.at[dslab.at[j]], sem).wait()

    def pair(i, carry):
        ja = 2 * i
        jb = 2 * i + 1

        @pl.when(i == 0)
        def _():
            sc_start(ja, semsc0)

        sc_start(jb, semsc1)
        sc_wait(ja, semsc0)

        @pl.when(i < CH // 2 - 1)
        def _():
            sc_start(ja + 2, semsc0)

        sc_wait(jb, semsc1)
        return carry

    lax.fori_loop(0, CH // 2, pair, 0)
    plsc.subcore_barrier()
    pltpu.sync_copy(deg_sh.at[pl.ds(r0, ROWS_PER_TILE)],
                    deg_out.at[c, pl.ds(r0, ROWS_PER_TILE)])


_sc_agg = pl.kernel(
    _sc_agg_body,
    out_type=jax.ShapeDtypeStruct((NC, N_PAD, D), jnp.float32),
    mesh=_mesh,
    scratch_types=[
        pltpu.VMEM((CHUNK,), jnp.int32),
        pltpu.VMEM((CHUNK,), jnp.int32),
        pltpu.VMEM((CHUNK, D), jnp.float32),
        pltpu.VMEM_SHARED((N_PAD, D), jnp.float32),
        pltpu.SemaphoreType.DMA,
    ],
)

_sc_deg = pl.kernel(
    _sc_deg_body,
    out_type=jax.ShapeDtypeStruct((NC, N_PAD, D), jnp.float32),
    mesh=_mesh,
    scratch_types=[
        pltpu.VMEM((CH, CHUNK), jnp.int32),
        pltpu.VMEM((CHUNK, D), jnp.float32),
        pltpu.VMEM_SHARED((N_PAD, D), jnp.float32),
        pltpu.SemaphoreType.DMA,
        pltpu.SemaphoreType.DMA,
    ],
)


ROW_BLOCK = 1000


def _tc_layer_body(relu, h_ref, a0_ref, a1_ref, d0_ref, d1_ref,
                   ws_ref, wn_ref, b_ref, o_ref):
    h = h_ref[...]
    agg = a0_ref[...] + a1_ref[...]
    deg = d0_ref[...][:, 0:1] + d1_ref[...][:, 0:1]
    h_neigh = agg / jnp.maximum(deg, 1.0)
    out = (jnp.dot(h, ws_ref[...], preferred_element_type=jnp.float32)
           + jnp.dot(h_neigh, wn_ref[...], preferred_element_type=jnp.float32)
           + b_ref[...])
    if relu:
        out = jnp.where(out >= 0, out, 0.01 * out)
    o_ref[...] = out


def _tc_layer(h, a0, a1, d0, d1, ws, wn, b, relu):
    grid = N // ROW_BLOCK
    row = pl.BlockSpec((ROW_BLOCK, D), lambda i: (i, 0))
    deg16 = pl.BlockSpec((ROW_BLOCK, 16), lambda i: (i, 0))
    full = pl.BlockSpec((D, D), lambda i: (0, 0))
    return pl.pallas_call(
        functools.partial(_tc_layer_body, relu),
        grid=(grid,),
        in_specs=[row, row, row, deg16, deg16, full, full,
                  pl.BlockSpec((1, D), lambda i: (0, 0))],
        out_specs=row,
        out_shape=jax.ShapeDtypeStruct((N, D), jnp.float32),
    )(h, a0, a1, d0, d1, ws, wn, b.reshape(1, D))


def kernel(in_feat, edge_index, Ws1, Wn1, b1, Ws2, Wn2, b2, Ws3, Wn3, b3):
    src = edge_index[0]
    dst = edge_index[1]
    pad = E_PAD - E
    src3 = jnp.concatenate(
        [src, jnp.zeros((pad,), jnp.int32)]).reshape(NW, CH, CHUNK)
    dst3 = jnp.concatenate(
        [dst, jnp.full((pad,), TRASH, jnp.int32)]).reshape(NW, CH, CHUNK)
    z128 = jnp.zeros((N_PAD, D), jnp.float32)
    ones128 = jnp.ones((CHUNK, D), jnp.float32)

    deg = _sc_deg(dst3, z128, ones128)
    agg1 = _sc_agg(in_feat, src3, dst3, z128)
    d0 = deg[0, :N, :16]
    d1 = deg[1, :N, :16]
    h1 = _tc_layer(in_feat, agg1[0, :N], agg1[1, :N], d0, d1,
                   Ws1, Wn1, b1, relu=True)
    agg2 = _sc_agg(h1, src3, dst3, z128)
    h2 = _tc_layer(h1, agg2[0, :N], agg2[1, :N], d0, d1,
                   Ws2, Wn2, b2, relu=True)
    agg3 = _sc_agg(h2, src3, dst3, z128)
    return _tc_layer(h2, agg3[0, :N], agg3[1, :N], d0, d1,
                     Ws3, Wn3, b3, relu=False)


# speedup vs baseline: 2.1676x; 2.1676x over previous
import functools

import jax
import jax.numpy as jnp
from jax import lax
from jax.experimental import pallas as pl
from jax.experimental.pallas import tpu as pltpu
from jax.experimental.pallas import tpu_sc as plsc

N = 10000
E = 320000
D = 128

NC = 2
NS = 16
NW = NC * NS
CHUNK = 128
CH = 80
E_PAD = NW * CH * CHUNK
PAIRS = CH // 2
N_PAD = 10112
ROWS_PER_TILE = N_PAD // NS
TRASH = N

_mesh = plsc.VectorSubcoreMesh(core_axis_name="c", subcore_axis_name="s")


def _sc_agg_body(h_hbm, src3, dst3, z128, agg_out,
                 sidx, didx, rows_v, agg_sh, sem):
    c = lax.axis_index("c")
    s = lax.axis_index("s")
    w = s * NC + c
    r0 = s * ROWS_PER_TILE
    pltpu.sync_copy(z128.at[pl.ds(r0, ROWS_PER_TILE)],
                    agg_sh.at[pl.ds(r0, ROWS_PER_TILE)])
    plsc.subcore_barrier()

    def chunk(j, carry):
        pltpu.sync_copy(src3.at[w, j], sidx)
        pltpu.sync_copy(dst3.at[w, j], didx)
        pltpu.async_copy(h_hbm.at[sidx], rows_v, sem).wait()
        pltpu.sync_copy(rows_v, agg_sh.at[didx], add=True)
        return carry

    lax.fori_loop(0, CH, chunk, 0)
    plsc.subcore_barrier()
    pltpu.sync_copy(agg_sh.at[pl.ds(r0, ROWS_PER_TILE)],
                    agg_out.at[c, pl.ds(r0, ROWS_PER_TILE)])


def _sc_deg_body(dst3, z128, ones_hbm, deg_out,
                 dslab, ones_v, deg_sh, semsc0, semsc1):
    c = lax.axis_index("c")
    s = lax.axis_index("s")
    w = s * NC + c
    r0 = s * ROWS_PER_TILE
    pltpu.sync_copy(z128.at[pl.ds(r0, ROWS_PER_TILE)],
                    deg_sh.at[pl.ds(r0, ROWS_PER_TILE)])
    pltpu.sync_copy(ones_hbm, ones_v)
    pltpu.sync_copy(dst3.at[w], dslab)
    plsc.subcore_barrier()

    def sc_start(j, sem):
        pltpu.async_copy(ones_v, deg_sh.at[dslab.at[j]], sem, add=True)

    def sc_wait(j, sem):
        pltpu.make_async_copy(ones_v, deg_sh.at[dslab.at[j]], sem).wait()

    def pair(i, carry):
        ja = 2 * i
        jb = 2 * i + 1

        @pl.when(i == 0)
        def _():
            sc_start(ja, semsc0)

        sc_start(jb, semsc1)
        sc_wait(ja, semsc0)

        @pl.when(i < CH // 2 - 1)
        def _():
            sc_start(ja + 2, semsc0)

        sc_wait(jb, semsc1)
        return carry

    lax.fori_loop(0, CH // 2, pair, 0)
    plsc.subcore_barrier()
    pltpu.sync_copy(deg_sh.at[pl.ds(r0, ROWS_PER_TILE)],
                    deg_out.at[c, pl.ds(r0, ROWS_PER_TILE)])


_sc_agg = pl.kernel(
    _sc_agg_body,
    out_type=jax.ShapeDtypeStruct((NC, N_PAD, D), jnp.float32),
    mesh=_mesh,
    scratch_types=[
        pltpu.VMEM((CHUNK,), jnp.int32),
        pltpu.VMEM((CHUNK,), jnp.int32),
        pltpu.VMEM((CHUNK, D), jnp.float32),
        pltpu.VMEM_SHARED((N_PAD, D), jnp.float32),
        pltpu.SemaphoreType.DMA,
    ],
)

_sc_deg = pl.kernel(
    _sc_deg_body,
    out_type=jax.ShapeDtypeStruct((NC, N_PAD, D), jnp.float32),
    mesh=_mesh,
    scratch_types=[
        pltpu.VMEM((CH, CHUNK), jnp.int32),
        pltpu.VMEM((CHUNK, D), jnp.float32),
        pltpu.VMEM_SHARED((N_PAD, D), jnp.float32),
        pltpu.SemaphoreType.DMA,
        pltpu.SemaphoreType.DMA,
    ],
)


ROW_BLOCK = 1000


def _tc_layer_body(relu, h_ref, a0_ref, a1_ref, d0_ref, d1_ref,
                   ws_ref, wn_ref, b_ref, o_ref):
    h = h_ref[...]
    agg = a0_ref[...] + a1_ref[...]
    deg = d0_ref[...][:, 0:1] + d1_ref[...][:, 0:1]
    h_neigh = agg / jnp.maximum(deg, 1.0)
    out = (jnp.dot(h, ws_ref[...], preferred_element_type=jnp.float32)
           + jnp.dot(h_neigh, wn_ref[...], preferred_element_type=jnp.float32)
           + b_ref[...])
    if relu:
        out = jnp.where(out >= 0, out, 0.01 * out)
    o_ref[...] = out


def _tc_layer(h, a0, a1, d0, d1, ws, wn, b, relu):
    grid = N // ROW_BLOCK
    row = pl.BlockSpec((ROW_BLOCK, D), lambda i: (i, 0))
    deg16 = pl.BlockSpec((ROW_BLOCK, 16), lambda i: (i, 0))
    full = pl.BlockSpec((D, D), lambda i: (0, 0))
    return pl.pallas_call(
        functools.partial(_tc_layer_body, relu),
        grid=(grid,),
        in_specs=[row, row, row, deg16, deg16, full, full,
                  pl.BlockSpec((1, D), lambda i: (0, 0))],
        out_specs=row,
        out_shape=jax.ShapeDtypeStruct((N, D), jnp.float32),
    )(h, a0, a1, d0, d1, ws, wn, b.reshape(1, D))


def kernel(in_feat, edge_index, Ws1, Wn1, b1, Ws2, Wn2, b2, Ws3, Wn3, b3):
    src = edge_index[0]
    dst = edge_index[1]
    pad = E_PAD - E
    pad_idx = jnp.arange(pad, dtype=jnp.int32)
    src3 = jnp.concatenate(
        [src, (pad_idx * 97) % N]).reshape(NW, CH, CHUNK)
    dst3 = jnp.concatenate(
        [dst, TRASH + pad_idx % (N_PAD - N)]).reshape(NW, CH, CHUNK)
    z128 = jnp.zeros((N_PAD, D), jnp.float32)
    ones128 = jnp.ones((CHUNK, D), jnp.float32)

    deg = _sc_deg(dst3, z128, ones128)
    agg1 = _sc_agg(in_feat, src3, dst3, z128)
    d0 = deg[0, :N, :16]
    d1 = deg[1, :N, :16]
    h1 = _tc_layer(in_feat, agg1[0, :N], agg1[1, :N], d0, d1,
                   Ws1, Wn1, b1, relu=True)
    agg2 = _sc_agg(h1, src3, dst3, z128)
    h2 = _tc_layer(h1, agg2[0, :N], agg2[1, :N], d0, d1,
                   Ws2, Wn2, b2, relu=True)
    agg3 = _sc_agg(h2, src3, dst3, z128)
    return _tc_layer(h2, agg3[0, :N], agg3[1, :N], d0, d1,
                     Ws3, Wn3, b3, relu=False)


# speedup vs baseline: 2.5798x; 1.1901x over previous
import functools

import jax
import jax.numpy as jnp
from jax import lax
from jax.experimental import pallas as pl
from jax.experimental.pallas import tpu as pltpu
from jax.experimental.pallas import tpu_sc as plsc

N = 10000
E = 320000
D = 128

NC = 2
NS = 16
NW = NC * NS
CHUNK = 128
CH = 80
E_PAD = NW * CH * CHUNK
PAIRS = CH // 2
N_PAD = 10112
ROWS_PER_TILE = N_PAD // NS
TRASH = N

_mesh = plsc.VectorSubcoreMesh(core_axis_name="c", subcore_axis_name="s")


def _sc_agg_body(h_hbm, src3, dst3, z128, agg_out,
                 sidx0, sidx1, didx0, didx1, rows0, rows1, agg_sh,
                 semg0, semg1, semsc0, semsc1):
    c = lax.axis_index("c")
    s = lax.axis_index("s")
    w = s * NC + c
    r0 = s * ROWS_PER_TILE
    pltpu.sync_copy(z128.at[pl.ds(r0, ROWS_PER_TILE)],
                    agg_sh.at[pl.ds(r0, ROWS_PER_TILE)])
    plsc.subcore_barrier()

    def fetch(j, sidx, didx):
        pltpu.sync_copy(src3.at[w, j], sidx)
        pltpu.sync_copy(dst3.at[w, j], didx)

    def pair(i, carry):
        jb = 2 * i + 1

        @pl.when(i == 0)
        def _():
            fetch(0, sidx0, didx0)
            pltpu.async_copy(h_hbm.at[sidx0], rows0, semg0)

        pltpu.make_async_copy(h_hbm.at[sidx0], rows0, semg0).wait()

        @pl.when(i > 0)
        def _():
            pltpu.make_async_copy(rows1, agg_sh.at[didx1], semsc1).wait()

        fetch(jb, sidx1, didx1)
        pltpu.async_copy(h_hbm.at[sidx1], rows1, semg1)
        pltpu.async_copy(rows0, agg_sh.at[didx0], semsc0, add=True)
        pltpu.make_async_copy(h_hbm.at[sidx1], rows1, semg1).wait()
        pltpu.make_async_copy(rows0, agg_sh.at[didx0], semsc0).wait()

        @pl.when(i < PAIRS - 1)
        def _():
            fetch(jb + 1, sidx0, didx0)
            pltpu.async_copy(h_hbm.at[sidx0], rows0, semg0)

        pltpu.async_copy(rows1, agg_sh.at[didx1], semsc1, add=True)
        return carry

    lax.fori_loop(0, PAIRS, pair, 0)
    pltpu.make_async_copy(rows1, agg_sh.at[didx1], semsc1).wait()
    plsc.subcore_barrier()
    pltpu.sync_copy(agg_sh.at[pl.ds(r0, ROWS_PER_TILE)],
                    agg_out.at[c, pl.ds(r0, ROWS_PER_TILE)])


def _sc_deg_body(dst3, z128, ones_hbm, deg_out,
                 dslab, ones_v, deg_sh, semsc0, semsc1):
    c = lax.axis_index("c")
    s = lax.axis_index("s")
    w = s * NC + c
    r0 = s * ROWS_PER_TILE
    pltpu.sync_copy(z128.at[pl.ds(r0, ROWS_PER_TILE)],
                    deg_sh.at[pl.ds(r0, ROWS_PER_TILE)])
    pltpu.sync_copy(ones_hbm, ones_v)
    pltpu.sync_copy(dst3.at[w], dslab)
    plsc.subcore_barrier()

    def sc_start(j, sem):
        pltpu.async_copy(ones_v, deg_sh.at[dslab.at[j]], sem, add=True)

    def sc_wait(j, sem):
        pltpu.make_async_copy(ones_v, deg_sh.at[dslab.at[j]], sem).wait()

    def pair(i, carry):
        ja = 2 * i
        jb = 2 * i + 1

        @pl.when(i == 0)
        def _():
            sc_start(ja, semsc0)

        sc_start(jb, semsc1)
        sc_wait(ja, semsc0)

        @pl.when(i < CH // 2 - 1)
        def _():
            sc_start(ja + 2, semsc0)

        sc_wait(jb, semsc1)
        return carry

    lax.fori_loop(0, CH // 2, pair, 0)
    plsc.subcore_barrier()
    pltpu.sync_copy(deg_sh.at[pl.ds(r0, ROWS_PER_TILE)],
                    deg_out.at[c, pl.ds(r0, ROWS_PER_TILE)])


_sc_agg = pl.kernel(
    _sc_agg_body,
    out_type=jax.ShapeDtypeStruct((NC, N_PAD, D), jnp.float32),
    mesh=_mesh,
    scratch_types=[
        pltpu.VMEM((CHUNK,), jnp.int32),
        pltpu.VMEM((CHUNK,), jnp.int32),
        pltpu.VMEM((CHUNK,), jnp.int32),
        pltpu.VMEM((CHUNK,), jnp.int32),
        pltpu.VMEM((CHUNK, D), jnp.float32),
        pltpu.VMEM((CHUNK, D), jnp.float32),
        pltpu.VMEM_SHARED((N_PAD, D), jnp.float32),
        pltpu.SemaphoreType.DMA,
        pltpu.SemaphoreType.DMA,
        pltpu.SemaphoreType.DMA,
        pltpu.SemaphoreType.DMA,
    ],
)

_sc_deg = pl.kernel(
    _sc_deg_body,
    out_type=jax.ShapeDtypeStruct((NC, N_PAD, D), jnp.float32),
    mesh=_mesh,
    scratch_types=[
        pltpu.VMEM((CH, CHUNK), jnp.int32),
        pltpu.VMEM((CHUNK, D), jnp.float32),
        pltpu.VMEM_SHARED((N_PAD, D), jnp.float32),
        pltpu.SemaphoreType.DMA,
        pltpu.SemaphoreType.DMA,
    ],
)


ROW_BLOCK = 1000


def _tc_layer_body(relu, h_ref, a0_ref, a1_ref, d0_ref, d1_ref,
                   ws_ref, wn_ref, b_ref, o_ref):
    h = h_ref[...]
    agg = a0_ref[...] + a1_ref[...]
    deg = d0_ref[...][:, 0:1] + d1_ref[...][:, 0:1]
    h_neigh = agg / jnp.maximum(deg, 1.0)
    out = (jnp.dot(h, ws_ref[...], preferred_element_type=jnp.float32)
           + jnp.dot(h_neigh, wn_ref[...], preferred_element_type=jnp.float32)
           + b_ref[...])
    if relu:
        out = jnp.where(out >= 0, out, 0.01 * out)
    o_ref[...] = out


def _tc_layer(h, a0, a1, d0, d1, ws, wn, b, relu):
    grid = N // ROW_BLOCK
    row = pl.BlockSpec((ROW_BLOCK, D), lambda i: (i, 0))
    deg16 = pl.BlockSpec((ROW_BLOCK, 16), lambda i: (i, 0))
    full = pl.BlockSpec((D, D), lambda i: (0, 0))
    return pl.pallas_call(
        functools.partial(_tc_layer_body, relu),
        grid=(grid,),
        in_specs=[row, row, row, deg16, deg16, full, full,
                  pl.BlockSpec((1, D), lambda i: (0, 0))],
        out_specs=row,
        out_shape=jax.ShapeDtypeStruct((N, D), jnp.float32),
    )(h, a0, a1, d0, d1, ws, wn, b.reshape(1, D))


def kernel(in_feat, edge_index, Ws1, Wn1, b1, Ws2, Wn2, b2, Ws3, Wn3, b3):
    src = edge_index[0]
    dst = edge_index[1]
    pad = E_PAD - E
    pad_idx = jnp.arange(pad, dtype=jnp.int32)
    src3 = jnp.concatenate(
        [src, (pad_idx * 97) % N]).reshape(NW, CH, CHUNK)
    dst3 = jnp.concatenate(
        [dst, TRASH + pad_idx % (N_PAD - N)]).reshape(NW, CH, CHUNK)
    z128 = jnp.zeros((N_PAD, D), jnp.float32)
    ones128 = jnp.ones((CHUNK, D), jnp.float32)

    deg = _sc_deg(dst3, z128, ones128)
    agg1 = _sc_agg(in_feat, src3, dst3, z128)
    d0 = deg[0, :N, :16]
    d1 = deg[1, :N, :16]
    h1 = _tc_layer(in_feat, agg1[0, :N], agg1[1, :N], d0, d1,
                   Ws1, Wn1, b1, relu=True)
    agg2 = _sc_agg(h1, src3, dst3, z128)
    h2 = _tc_layer(h1, agg2[0, :N], agg2[1, :N], d0, d1,
                   Ws2, Wn2, b2, relu=True)
    agg3 = _sc_agg(h2, src3, dst3, z128)
    return _tc_layer(h2, agg3[0, :N], agg3[1, :N], d0, d1,
                     Ws3, Wn3, b3, relu=False)
